# Initial kernel scaffold; baseline (speedup 1.0000x reference)
import jax
import jax.numpy as jnp
from jax.experimental import pallas as pl


def kernel(x_user, x_item, edge_index_ui, edge_index_iu, W_enc_user, b_enc_user, W_enc_item, b_enc_item, Wl0_ui, bl0_ui, Wr0_ui, Wl0_iu, bl0_iu, Wr0_iu, Wl1_ui, bl1_ui, Wr1_ui, Wl1_iu, bl1_iu, Wr1_iu):
    raise NotImplementedError("write your pallas kernel here")


# speedup vs baseline: 3.7774x; 3.7774x over previous
import jax
import jax.numpy as jnp
from jax import lax
from jax.experimental import pallas as pl
from jax.experimental.pallas import tpu as pltpu
from jax.experimental.pallas import tpu_sc as plsc

N = 10000
D = 128
E = 320000
CNTW = 16
CH = 80
NSUB = 16
NCHUNK = E // CH // NSUB
EPT = E // NSUB
STRIPE = 624
TAIL = N - NSUB * STRIPE
STG = 48


def _zero_accum(accum, stage, sid):
  zero16 = jnp.zeros((16,), jnp.float32)
  def zs(i, _):
    for j in range(D // 16):
      stage[i, pl.ds(j * 16, 16)] = zero16
    return 0
  lax.fori_loop(0, STG, zs, 0)
  def za(i, _):
    pltpu.sync_copy(stage, accum.at[pl.ds(sid * STRIPE + i * STG, STG)])
    return 0
  lax.fori_loop(0, STRIPE // STG, za, 0)
  @pl.when(sid == NSUB - 1)
  def _():
    pltpu.sync_copy(stage.at[pl.ds(0, TAIL)],
                    accum.at[pl.ds(NSUB * STRIPE, TAIL)])


def _write_out(accum, stage, sid, out):
  def wo(i, _):
    sl = pl.ds(sid * STRIPE + i * STG, STG)
    pltpu.sync_copy(accum.at[sl], stage)
    pltpu.sync_copy(stage, out.at[sl])
    return 0
  lax.fori_loop(0, STRIPE // STG, wo, 0)
  @pl.when(sid == NSUB - 1)
  def _():
    tl = pl.ds(NSUB * STRIPE, TAIL)
    pltpu.sync_copy(accum.at[tl], stage.at[pl.ds(0, TAIL)])
    pltpu.sync_copy(stage.at[pl.ds(0, TAIL)], out.at[tl])


def _agg_body(hA, srcA, dstA, hB, srcB, dstB,
              sumA, sumB, accum, src_b, dst_b, rows, stage, rsem, wsem):
  cid = lax.axis_index("c")
  sid = lax.axis_index("s")
  _zero_accum(accum, stage, sid)
  plsc.subcore_barrier()

  def run(src_hbm, dst_hbm, h, sum_out):
    base = sid * EPT
    def step(g, _):
      off = base + g * CH
      pltpu.sync_copy(src_hbm.at[pl.ds(off, CH)], src_b)
      pltpu.sync_copy(dst_hbm.at[pl.ds(off, CH)], dst_b)
      pltpu.async_copy(h.at[src_b], rows, rsem).wait()
      pltpu.async_copy(rows, accum.at[dst_b], wsem, add=True).wait()
      return 0
    lax.fori_loop(0, NCHUNK, step, 0)
    plsc.subcore_barrier()
    _write_out(accum, stage, sid, sum_out)

  @pl.when(cid == 0)
  def _():
    run(srcA, dstA, hA, sumA)

  @pl.when(cid == 1)
  def _():
    run(srcB, dstB, hB, sumB)


def _cnt_body(dstA, dstB, cntA, cntB,
              accum, dst_b, ones_v, stage, wsem):
  cid = lax.axis_index("c")
  sid = lax.axis_index("s")
  _zero_accum(accum, stage, sid)
  one16 = jnp.ones((16,), jnp.float32)
  def ob(i, _):
    for j in range(D // 16):
      ones_v[i, pl.ds(j * 16, 16)] = one16
    return 0
  lax.fori_loop(0, CH, ob, 0)
  plsc.subcore_barrier()

  def run(dst_hbm, cnt_out):
    base = sid * EPT
    def step(g, _):
      off = base + g * CH
      pltpu.sync_copy(dst_hbm.at[pl.ds(off, CH)], dst_b)
      pltpu.async_copy(ones_v, accum.at[dst_b], wsem, add=True).wait()
      return 0
    lax.fori_loop(0, NCHUNK, step, 0)
    plsc.subcore_barrier()
    _write_out(accum, stage, sid, cnt_out)

  @pl.when(cid == 0)
  def _():
    run(dstA, cntA)

  @pl.when(cid == 1)
  def _():
    run(dstB, cntB)


_SC_MESH = plsc.VectorSubcoreMesh(core_axis_name="c", subcore_axis_name="s")

_agg = pl.kernel(
    _agg_body,
    out_type=(jax.ShapeDtypeStruct((N, D), jnp.float32),
              jax.ShapeDtypeStruct((N, D), jnp.float32)),
    mesh=_SC_MESH,
    scratch_types=(
        pltpu.VMEM_SHARED((N, D), jnp.float32),
        pltpu.VMEM((CH,), jnp.int32),
        pltpu.VMEM((CH,), jnp.int32),
        pltpu.VMEM((CH, D), jnp.float32),
        pltpu.VMEM((STG, D), jnp.float32),
        pltpu.SemaphoreType.DMA,
        pltpu.SemaphoreType.DMA,
    ))

_cnt = pl.kernel(
    _cnt_body,
    out_type=(jax.ShapeDtypeStruct((N, D), jnp.float32),
              jax.ShapeDtypeStruct((N, D), jnp.float32)),
    mesh=_SC_MESH,
    scratch_types=(
        pltpu.VMEM_SHARED((N, D), jnp.float32),
        pltpu.VMEM((CH,), jnp.int32),
        pltpu.VMEM((CH, D), jnp.float32),
        pltpu.VMEM((STG, D), jnp.float32),
        pltpu.SemaphoreType.DMA,
    ))


BR = 400


def _enc_body(x_ref, w_ref, b_ref, o_ref):
  o_ref[...] = (jnp.dot(x_ref[...], w_ref[...],
                        preferred_element_type=jnp.float32) + b_ref[...])


def _enc(x, W, b):
  return pl.pallas_call(
      _enc_body,
      grid=(N // BR,),
      in_specs=[pl.BlockSpec((BR, D), lambda i: (i, 0)),
                pl.BlockSpec((D, D), lambda i: (0, 0)),
                pl.BlockSpec((1, D), lambda i: (0, 0))],
      out_specs=pl.BlockSpec((BR, D), lambda i: (i, 0)),
      out_shape=jax.ShapeDtypeStruct((N, D), jnp.float32),
  )(x, W, b.reshape(1, D))


def _layer_body(s_ref, c_ref, h_ref, wl_ref, bl_ref, wr_ref, o_ref):
  mean = s_ref[...] / jnp.maximum(c_ref[...], 1.0)
  o_ref[...] = (jnp.dot(mean, wl_ref[...], preferred_element_type=jnp.float32)
                + jnp.dot(h_ref[...], wr_ref[...],
                          preferred_element_type=jnp.float32)
                + bl_ref[...])


def _layer(s, cnt_col, h, Wl, bl, Wr):
  return pl.pallas_call(
      _layer_body,
      grid=(N // BR,),
      in_specs=[pl.BlockSpec((BR, D), lambda i: (i, 0)),
                pl.BlockSpec((BR, 1), lambda i: (i, 0)),
                pl.BlockSpec((BR, D), lambda i: (i, 0)),
                pl.BlockSpec((D, D), lambda i: (0, 0)),
                pl.BlockSpec((1, D), lambda i: (0, 0)),
                pl.BlockSpec((D, D), lambda i: (0, 0))],
      out_specs=pl.BlockSpec((BR, D), lambda i: (i, 0)),
      out_shape=jax.ShapeDtypeStruct((N, D), jnp.float32),
  )(s, cnt_col, h, Wl, bl.reshape(1, D), Wr)


def kernel(x_user, x_item, edge_index_ui, edge_index_iu,
           W_enc_user, b_enc_user, W_enc_item, b_enc_item,
           Wl0_ui, bl0_ui, Wr0_ui, Wl0_iu, bl0_iu, Wr0_iu,
           Wl1_ui, bl1_ui, Wr1_ui, Wl1_iu, bl1_iu, Wr1_iu):
  hu = _enc(x_user, W_enc_user, b_enc_user)
  hi = _enc(x_item, W_enc_item, b_enc_item)

  src_ui = edge_index_ui[0]
  dst_ui = edge_index_ui[1]
  src_iu = edge_index_iu[0]
  dst_iu = edge_index_iu[1]

  c_ui, c_iu = _cnt(dst_ui, dst_iu)
  cu = c_ui[:, :1]
  ci = c_iu[:, :1]

  s_ui, s_iu = _agg(hu, src_ui, dst_ui, hi, src_iu, dst_iu)
  ni = _layer(s_ui, cu, hi, Wl0_ui, bl0_ui, Wr0_ui)
  nu = _layer(s_iu, ci, hu, Wl0_iu, bl0_iu, Wr0_iu)

  s_ui1, s_iu1 = _agg(nu, src_ui, dst_ui, ni, src_iu, dst_iu)
  ni2 = _layer(s_ui1, cu, ni, Wl1_ui, bl1_ui, Wr1_ui)
  nu2 = _layer(s_iu1, ci, nu, Wl1_iu, bl1_iu, Wr1_iu)
  return (nu2, ni2)


# speedup vs baseline: 8.1507x; 2.1577x over previous
import jax
import jax.numpy as jnp
from jax import lax
from jax.experimental import pallas as pl
from jax.experimental.pallas import tpu as pltpu
from jax.experimental.pallas import tpu_sc as plsc

N = 10000
D = 128
E = 320000
CH = 128
NSUB = 16
NCT = E // CH
ITERS = -(-NCT // NSUB)
STRIPE = 624
TAIL = N - NSUB * STRIPE
STG = 48


def _zero_accum(accum, stage, sid):
  zero16 = jnp.zeros((16,), jnp.float32)
  def zs(i, _):
    for j in range(D // 16):
      stage[i, pl.ds(j * 16, 16)] = zero16
    return 0
  lax.fori_loop(0, STG, zs, 0)
  def za(i, _):
    pltpu.sync_copy(stage, accum.at[pl.ds(sid * STRIPE + i * STG, STG)])
    return 0
  lax.fori_loop(0, STRIPE // STG, za, 0)
  @pl.when(sid == NSUB - 1)
  def _():
    pltpu.sync_copy(stage.at[pl.ds(0, TAIL)],
                    accum.at[pl.ds(NSUB * STRIPE, TAIL)])


def _write_out(accum, stage, sid, out):
  def wo(i, _):
    sl = pl.ds(sid * STRIPE + i * STG, STG)
    pltpu.sync_copy(accum.at[sl], stage)
    pltpu.sync_copy(stage, out.at[sl])
    return 0
  lax.fori_loop(0, STRIPE // STG, wo, 0)
  @pl.when(sid == NSUB - 1)
  def _():
    tl = pl.ds(NSUB * STRIPE, TAIL)
    pltpu.sync_copy(accum.at[tl], stage.at[pl.ds(0, TAIL)])
    pltpu.sync_copy(stage.at[pl.ds(0, TAIL)], out.at[tl])


def _agg_body(hA, eiA, hB, eiB,
              sumA, sumB, accum, idx0, idx1, rows0, rows1, stage,
              isem0, isem1, gsem0, gsem1, wsem):
  cid = lax.axis_index("c")
  sid = lax.axis_index("s")
  _zero_accum(accum, stage, sid)
  plsc.subcore_barrier()

  idxs = (idx0, idx1)
  isems = (isem0, isem1)
  rowss = (rows0, rows1)
  gsems = (gsem0, gsem1)

  def run(ei, h, sum_out):
    def active(i):
      return sid + NSUB * i < NCT

    def start_idx(b, i):
      off = (sid + NSUB * i) * CH
      pltpu.make_async_copy(ei.at[:, pl.ds(off, CH)], idxs[b],
                            isems[b]).start()

    def wait_idx(b):
      pltpu.make_async_copy(ei.at[:, pl.ds(0, CH)], idxs[b], isems[b]).wait()

    def start_gather(b):
      pltpu.make_async_copy(h.at[idxs[b].at[0]], rowss[b], gsems[b]).start()

    def wait_gather(b):
      pltpu.make_async_copy(h.at[idxs[b].at[0]], rowss[b], gsems[b]).wait()

    start_idx(0, 0)
    start_idx(1, 1)
    wait_idx(0)
    start_gather(0)

    def step(k, _):
      for b in range(2):
        i = 2 * k + b
        o = 1 - b
        @pl.when(active(i))
        def _():
          wait_gather(b)
        @pl.when(active(i + 1))
        def _():
          wait_idx(o)
          start_gather(o)
        @pl.when(active(i))
        def _():
          pltpu.async_copy(rowss[b], accum.at[idxs[b].at[1]], wsem,
                           add=True).wait()
        @pl.when(active(i + 2))
        def _():
          start_idx(b, i + 2)
      return 0
    lax.fori_loop(0, (ITERS + 1) // 2, step, 0)

    plsc.subcore_barrier()
    _write_out(accum, stage, sid, sum_out)

  @pl.when(cid == 0)
  def _():
    run(eiA, hA, sumA)

  @pl.when(cid == 1)
  def _():
    run(eiB, hB, sumB)


def _cnt_body(eiA, eiB, cntA, cntB,
              accum, idx0, idx1, ones_v, stage, isem0, isem1, wsem):
  cid = lax.axis_index("c")
  sid = lax.axis_index("s")
  _zero_accum(accum, stage, sid)
  one16 = jnp.ones((16,), jnp.float32)
  def ob(i, _):
    for j in range(D // 16):
      ones_v[i, pl.ds(j * 16, 16)] = one16
    return 0
  lax.fori_loop(0, CH, ob, 0)
  plsc.subcore_barrier()

  idxs = (idx0, idx1)
  isems = (isem0, isem1)

  def run(ei, cnt_out):
    def active(i):
      return sid + NSUB * i < NCT

    def start_idx(b, i):
      off = (sid + NSUB * i) * CH
      pltpu.make_async_copy(ei.at[:, pl.ds(off, CH)], idxs[b],
                            isems[b]).start()

    def wait_idx(b):
      pltpu.make_async_copy(ei.at[:, pl.ds(0, CH)], idxs[b], isems[b]).wait()

    start_idx(0, 0)
    start_idx(1, 1)

    def step(k, _):
      for b in range(2):
        i = 2 * k + b
        @pl.when(active(i))
        def _():
          wait_idx(b)
          pltpu.async_copy(ones_v, accum.at[idxs[b].at[1]], wsem,
                           add=True).wait()
        @pl.when(active(i + 2))
        def _():
          start_idx(b, i + 2)
      return 0
    lax.fori_loop(0, (ITERS + 1) // 2, step, 0)

    plsc.subcore_barrier()
    _write_out(accum, stage, sid, cnt_out)

  @pl.when(cid == 0)
  def _():
    run(eiA, cntA)

  @pl.when(cid == 1)
  def _():
    run(eiB, cntB)


_SC_MESH = plsc.VectorSubcoreMesh(core_axis_name="c", subcore_axis_name="s")

_agg = pl.kernel(
    _agg_body,
    out_type=(jax.ShapeDtypeStruct((N, D), jnp.float32),
              jax.ShapeDtypeStruct((N, D), jnp.float32)),
    mesh=_SC_MESH,
    scratch_types=(
        pltpu.VMEM_SHARED((N, D), jnp.float32),
        pltpu.VMEM((2, CH), jnp.int32),
        pltpu.VMEM((2, CH), jnp.int32),
        pltpu.VMEM((CH, D), jnp.float32),
        pltpu.VMEM((CH, D), jnp.float32),
        pltpu.VMEM((STG, D), jnp.float32),
        pltpu.SemaphoreType.DMA,
        pltpu.SemaphoreType.DMA,
        pltpu.SemaphoreType.DMA,
        pltpu.SemaphoreType.DMA,
        pltpu.SemaphoreType.DMA,
    ))

_cnt = pl.kernel(
    _cnt_body,
    out_type=(jax.ShapeDtypeStruct((N, D), jnp.float32),
              jax.ShapeDtypeStruct((N, D), jnp.float32)),
    mesh=_SC_MESH,
    scratch_types=(
        pltpu.VMEM_SHARED((N, D), jnp.float32),
        pltpu.VMEM((2, CH), jnp.int32),
        pltpu.VMEM((2, CH), jnp.int32),
        pltpu.VMEM((CH, D), jnp.float32),
        pltpu.VMEM((STG, D), jnp.float32),
        pltpu.SemaphoreType.DMA,
        pltpu.SemaphoreType.DMA,
        pltpu.SemaphoreType.DMA,
    ))


BR = 400


def _enc_body(x_ref, w_ref, b_ref, o_ref):
  o_ref[...] = (jnp.dot(x_ref[...], w_ref[...],
                        preferred_element_type=jnp.float32) + b_ref[...])


def _enc(x, W, b):
  return pl.pallas_call(
      _enc_body,
      grid=(N // BR,),
      in_specs=[pl.BlockSpec((BR, D), lambda i: (i, 0)),
                pl.BlockSpec((D, D), lambda i: (0, 0)),
                pl.BlockSpec((1, D), lambda i: (0, 0))],
      out_specs=pl.BlockSpec((BR, D), lambda i: (i, 0)),
      out_shape=jax.ShapeDtypeStruct((N, D), jnp.float32),
  )(x, W, b.reshape(1, D))


def _layer_body(s_ref, c_ref, h_ref, wl_ref, bl_ref, wr_ref, o_ref):
  mean = s_ref[...] / jnp.maximum(c_ref[...], 1.0)
  o_ref[...] = (jnp.dot(mean, wl_ref[...], preferred_element_type=jnp.float32)
                + jnp.dot(h_ref[...], wr_ref[...],
                          preferred_element_type=jnp.float32)
                + bl_ref[...])


def _layer(s, cnt_col, h, Wl, bl, Wr):
  return pl.pallas_call(
      _layer_body,
      grid=(N // BR,),
      in_specs=[pl.BlockSpec((BR, D), lambda i: (i, 0)),
                pl.BlockSpec((BR, 1), lambda i: (i, 0)),
                pl.BlockSpec((BR, D), lambda i: (i, 0)),
                pl.BlockSpec((D, D), lambda i: (0, 0)),
                pl.BlockSpec((1, D), lambda i: (0, 0)),
                pl.BlockSpec((D, D), lambda i: (0, 0))],
      out_specs=pl.BlockSpec((BR, D), lambda i: (i, 0)),
      out_shape=jax.ShapeDtypeStruct((N, D), jnp.float32),
  )(s, cnt_col, h, Wl, bl.reshape(1, D), Wr)


def kernel(x_user, x_item, edge_index_ui, edge_index_iu,
           W_enc_user, b_enc_user, W_enc_item, b_enc_item,
           Wl0_ui, bl0_ui, Wr0_ui, Wl0_iu, bl0_iu, Wr0_iu,
           Wl1_ui, bl1_ui, Wr1_ui, Wl1_iu, bl1_iu, Wr1_iu):
  hu = _enc(x_user, W_enc_user, b_enc_user)
  hi = _enc(x_item, W_enc_item, b_enc_item)

  c_ui, c_iu = _cnt(edge_index_ui, edge_index_iu)
  cu = c_ui[:, :1]
  ci = c_iu[:, :1]

  s_ui, s_iu = _agg(hu, edge_index_ui, hi, edge_index_iu)
  ni = _layer(s_ui, cu, hi, Wl0_ui, bl0_ui, Wr0_ui)
  nu = _layer(s_iu, ci, hu, Wl0_iu, bl0_iu, Wr0_iu)

  s_ui1, s_iu1 = _agg(nu, edge_index_ui, ni, edge_index_iu)
  ni2 = _layer(s_ui1, cu, ni, Wl1_ui, bl1_ui, Wr1_ui)
  nu2 = _layer(s_iu1, ci, nu, Wl1_iu, bl1_iu, Wr1_iu)
  return (nu2, ni2)


# speedup vs baseline: 8.1972x; 1.0057x over previous
import jax
import jax.numpy as jnp
from jax import lax
from jax.experimental import pallas as pl
from jax.experimental.pallas import tpu as pltpu
from jax.experimental.pallas import tpu_sc as plsc

N = 10000
D = 128
E = 320000
CH = 128
NSUB = 16
NCT = E // CH
ITERS = -(-NCT // NSUB)
STRIPE = 624
TAIL = N - NSUB * STRIPE
STG = 48


def _zero_accum(accum, stage, sid):
  zero16 = jnp.zeros((16,), jnp.float32)
  def zs(i, _):
    for j in range(D // 16):
      stage[i, pl.ds(j * 16, 16)] = zero16
    return 0
  lax.fori_loop(0, STG, zs, 0)
  def za(i, _):
    pltpu.sync_copy(stage, accum.at[pl.ds(sid * STRIPE + i * STG, STG)])
    return 0
  lax.fori_loop(0, STRIPE // STG, za, 0)
  @pl.when(sid == NSUB - 1)
  def _():
    pltpu.sync_copy(stage.at[pl.ds(0, TAIL)],
                    accum.at[pl.ds(NSUB * STRIPE, TAIL)])


def _write_out(accum, stage, sid, out):
  def wo(i, _):
    sl = pl.ds(sid * STRIPE + i * STG, STG)
    pltpu.sync_copy(accum.at[sl], stage)
    pltpu.sync_copy(stage, out.at[sl])
    return 0
  lax.fori_loop(0, STRIPE // STG, wo, 0)
  @pl.when(sid == NSUB - 1)
  def _():
    tl = pl.ds(NSUB * STRIPE, TAIL)
    pltpu.sync_copy(accum.at[tl], stage.at[pl.ds(0, TAIL)])
    pltpu.sync_copy(stage.at[pl.ds(0, TAIL)], out.at[tl])


def _agg_body(hA, eiA, hB, eiB,
              sumA, sumB, accum, idx0, idx1, idx2, idx3, rows0, rows1, stage,
              isem0, isem1, isem2, isem3, gsem0, gsem1, wsem0, wsem1):
  cid = lax.axis_index("c")
  sid = lax.axis_index("s")
  _zero_accum(accum, stage, sid)
  plsc.subcore_barrier()

  idxs = (idx0, idx1, idx2, idx3)
  isems = (isem0, isem1, isem2, isem3)
  rowss = (rows0, rows1)
  gsems = (gsem0, gsem1)
  wsems = (wsem0, wsem1)

  def run(ei, h, sum_out):
    def active(i):
      return sid + NSUB * i < NCT

    def start_idx(q, i):
      off = (sid + NSUB * i) * CH
      pltpu.make_async_copy(ei.at[:, pl.ds(off, CH)], idxs[q],
                            isems[q]).start()

    def wait_idx(q):
      pltpu.make_async_copy(ei.at[:, pl.ds(0, CH)], idxs[q], isems[q]).wait()

    def start_gather(b, q):
      pltpu.make_async_copy(h.at[idxs[q].at[0]], rowss[b], gsems[b]).start()

    def wait_gather(b, q):
      pltpu.make_async_copy(h.at[idxs[q].at[0]], rowss[b], gsems[b]).wait()

    def start_scatter(b, q):
      pltpu.make_async_copy(rowss[b], accum.at[idxs[q].at[1]],
                            wsems[b]).start(add=True)

    def wait_scatter(b, q):
      pltpu.make_async_copy(rowss[b], accum.at[idxs[q].at[1]],
                            wsems[b]).wait()

    start_idx(0, 0)
    start_idx(1, 1)
    wait_idx(0)
    start_gather(0, 0)

    def step(k, _):
      for b4 in range(4):
        j = 4 * k + b4
        b = b4 % 2
        o = 1 - b
        qj = b4
        qp = (b4 - 1) % 4
        qn = (b4 + 1) % 4
        qn2 = (b4 + 2) % 4
        @pl.when(active(j))
        def _():
          wait_gather(b, qj)
          start_scatter(b, qj)
        @pl.when((j >= 1) & active(j - 1))
        def _():
          wait_scatter(o, qp)
        @pl.when(active(j + 1))
        def _():
          wait_idx(qn)
          start_gather(o, qn)
        @pl.when(active(j + 2))
        def _():
          start_idx(qn2, j + 2)
      return 0
    lax.fori_loop(0, (ITERS + 4) // 4, step, 0)

    plsc.subcore_barrier()
    _write_out(accum, stage, sid, sum_out)

  @pl.when(cid == 0)
  def _():
    run(eiA, hA, sumA)

  @pl.when(cid == 1)
  def _():
    run(eiB, hB, sumB)


def _cnt_body(eiA, eiB, cntA, cntB,
              accum, idx0, idx1, idx2, idx3, ones_v, stage,
              isem0, isem1, isem2, isem3, wsem0, wsem1):
  cid = lax.axis_index("c")
  sid = lax.axis_index("s")
  _zero_accum(accum, stage, sid)
  one16 = jnp.ones((16,), jnp.float32)
  def ob(i, _):
    for j in range(D // 16):
      ones_v[i, pl.ds(j * 16, 16)] = one16
    return 0
  lax.fori_loop(0, CH, ob, 0)
  plsc.subcore_barrier()

  idxs = (idx0, idx1, idx2, idx3)
  isems = (isem0, isem1, isem2, isem3)
  wsems = (wsem0, wsem1)

  def run(ei, cnt_out):
    def active(i):
      return sid + NSUB * i < NCT

    def start_idx(q, i):
      off = (sid + NSUB * i) * CH
      pltpu.make_async_copy(ei.at[:, pl.ds(off, CH)], idxs[q],
                            isems[q]).start()

    def wait_idx(q):
      pltpu.make_async_copy(ei.at[:, pl.ds(0, CH)], idxs[q], isems[q]).wait()

    def start_scatter(b, q):
      pltpu.make_async_copy(ones_v, accum.at[idxs[q].at[1]],
                            wsems[b]).start(add=True)

    def wait_scatter(b, q):
      pltpu.make_async_copy(ones_v, accum.at[idxs[q].at[1]],
                            wsems[b]).wait()

    start_idx(0, 0)
    start_idx(1, 1)

    def step(k, _):
      for b4 in range(4):
        j = 4 * k + b4
        b = b4 % 2
        o = 1 - b
        qj = b4
        qp = (b4 - 1) % 4
        qn2 = (b4 + 2) % 4
        @pl.when(active(j))
        def _():
          wait_idx(qj)
          start_scatter(b, qj)
        @pl.when((j >= 1) & active(j - 1))
        def _():
          wait_scatter(o, qp)
        @pl.when(active(j + 2))
        def _():
          start_idx(qn2, j + 2)
      return 0
    lax.fori_loop(0, (ITERS + 4) // 4, step, 0)

    plsc.subcore_barrier()
    _write_out(accum, stage, sid, cnt_out)

  @pl.when(cid == 0)
  def _():
    run(eiA, cntA)

  @pl.when(cid == 1)
  def _():
    run(eiB, cntB)


_SC_MESH = plsc.VectorSubcoreMesh(core_axis_name="c", subcore_axis_name="s")

_agg = pl.kernel(
    _agg_body,
    out_type=(jax.ShapeDtypeStruct((N, D), jnp.float32),
              jax.ShapeDtypeStruct((N, D), jnp.float32)),
    mesh=_SC_MESH,
    scratch_types=(
        pltpu.VMEM_SHARED((N, D), jnp.float32),
        pltpu.VMEM((2, CH), jnp.int32),
        pltpu.VMEM((2, CH), jnp.int32),
        pltpu.VMEM((2, CH), jnp.int32),
        pltpu.VMEM((2, CH), jnp.int32),
        pltpu.VMEM((CH, D), jnp.float32),
        pltpu.VMEM((CH, D), jnp.float32),
        pltpu.VMEM((STG, D), jnp.float32),
        pltpu.SemaphoreType.DMA,
        pltpu.SemaphoreType.DMA,
        pltpu.SemaphoreType.DMA,
        pltpu.SemaphoreType.DMA,
        pltpu.SemaphoreType.DMA,
        pltpu.SemaphoreType.DMA,
        pltpu.SemaphoreType.DMA,
        pltpu.SemaphoreType.DMA,
    ))

_cnt = pl.kernel(
    _cnt_body,
    out_type=(jax.ShapeDtypeStruct((N, D), jnp.float32),
              jax.ShapeDtypeStruct((N, D), jnp.float32)),
    mesh=_SC_MESH,
    scratch_types=(
        pltpu.VMEM_SHARED((N, D), jnp.float32),
        pltpu.VMEM((2, CH), jnp.int32),
        pltpu.VMEM((2, CH), jnp.int32),
        pltpu.VMEM((2, CH), jnp.int32),
        pltpu.VMEM((2, CH), jnp.int32),
        pltpu.VMEM((CH, D), jnp.float32),
        pltpu.VMEM((STG, D), jnp.float32),
        pltpu.SemaphoreType.DMA,
        pltpu.SemaphoreType.DMA,
        pltpu.SemaphoreType.DMA,
        pltpu.SemaphoreType.DMA,
        pltpu.SemaphoreType.DMA,
        pltpu.SemaphoreType.DMA,
    ))


BR = 400


def _enc_body(x_ref, w_ref, b_ref, o_ref):
  o_ref[...] = (jnp.dot(x_ref[...], w_ref[...],
                        preferred_element_type=jnp.float32) + b_ref[...])


def _enc(x, W, b):
  return pl.pallas_call(
      _enc_body,
      grid=(N // BR,),
      in_specs=[pl.BlockSpec((BR, D), lambda i: (i, 0)),
                pl.BlockSpec((D, D), lambda i: (0, 0)),
                pl.BlockSpec((1, D), lambda i: (0, 0))],
      out_specs=pl.BlockSpec((BR, D), lambda i: (i, 0)),
      out_shape=jax.ShapeDtypeStruct((N, D), jnp.float32),
  )(x, W, b.reshape(1, D))


def _layer_body(s_ref, c_ref, h_ref, wl_ref, bl_ref, wr_ref, o_ref):
  mean = s_ref[...] / jnp.maximum(c_ref[...], 1.0)
  o_ref[...] = (jnp.dot(mean, wl_ref[...], preferred_element_type=jnp.float32)
                + jnp.dot(h_ref[...], wr_ref[...],
                          preferred_element_type=jnp.float32)
                + bl_ref[...])


def _layer(s, cnt_col, h, Wl, bl, Wr):
  return pl.pallas_call(
      _layer_body,
      grid=(N // BR,),
      in_specs=[pl.BlockSpec((BR, D), lambda i: (i, 0)),
                pl.BlockSpec((BR, 1), lambda i: (i, 0)),
                pl.BlockSpec((BR, D), lambda i: (i, 0)),
                pl.BlockSpec((D, D), lambda i: (0, 0)),
                pl.BlockSpec((1, D), lambda i: (0, 0)),
                pl.BlockSpec((D, D), lambda i: (0, 0))],
      out_specs=pl.BlockSpec((BR, D), lambda i: (i, 0)),
      out_shape=jax.ShapeDtypeStruct((N, D), jnp.float32),
  )(s, cnt_col, h, Wl, bl.reshape(1, D), Wr)


def kernel(x_user, x_item, edge_index_ui, edge_index_iu,
           W_enc_user, b_enc_user, W_enc_item, b_enc_item,
           Wl0_ui, bl0_ui, Wr0_ui, Wl0_iu, bl0_iu, Wr0_iu,
           Wl1_ui, bl1_ui, Wr1_ui, Wl1_iu, bl1_iu, Wr1_iu):
  hu = _enc(x_user, W_enc_user, b_enc_user)
  hi = _enc(x_item, W_enc_item, b_enc_item)

  c_ui, c_iu = _cnt(edge_index_ui, edge_index_iu)
  cu = c_ui[:, :1]
  ci = c_iu[:, :1]

  s_ui, s_iu = _agg(hu, edge_index_ui, hi, edge_index_iu)
  ni = _layer(s_ui, cu, hi, Wl0_ui, bl0_ui, Wr0_ui)
  nu = _layer(s_iu, ci, hu, Wl0_iu, bl0_iu, Wr0_iu)

  s_ui1, s_iu1 = _agg(nu, edge_index_ui, ni, edge_index_iu)
  ni2 = _layer(s_ui1, cu, ni, Wl1_ui, bl1_ui, Wr1_ui)
  nu2 = _layer(s_iu1, ci, nu, Wl1_iu, bl1_iu, Wr1_iu)
  return (nu2, ni2)


# speedup vs baseline: 9.3473x; 1.1403x over previous
import jax
import jax.numpy as jnp
from jax import lax
from jax.experimental import pallas as pl
from jax.experimental.pallas import tpu as pltpu
from jax.experimental.pallas import tpu_sc as plsc

N = 10000
D = 128
E = 320000
CH = 128
NSUB = 16
NCT = E // CH
ITERS = -(-NCT // NSUB)
STRIPE = 624
TAIL = N - NSUB * STRIPE
STG = 48


def _zero_accum(accum, stage, sid):
  zero16 = jnp.zeros((16,), jnp.float32)
  def zs(i, _):
    for j in range(D // 16):
      stage[i, pl.ds(j * 16, 16)] = zero16
    return 0
  lax.fori_loop(0, STG, zs, 0)
  def za(i, _):
    pltpu.sync_copy(stage, accum.at[pl.ds(sid * STRIPE + i * STG, STG)])
    return 0
  lax.fori_loop(0, STRIPE // STG, za, 0)
  @pl.when(sid == NSUB - 1)
  def _():
    pltpu.sync_copy(stage.at[pl.ds(0, TAIL)],
                    accum.at[pl.ds(NSUB * STRIPE, TAIL)])


def _write_out(accum, stage, sid, out):
  def wo(i, _):
    sl = pl.ds(sid * STRIPE + i * STG, STG)
    pltpu.sync_copy(accum.at[sl], stage)
    pltpu.sync_copy(stage, out.at[sl])
    return 0
  lax.fori_loop(0, STRIPE // STG, wo, 0)
  @pl.when(sid == NSUB - 1)
  def _():
    tl = pl.ds(NSUB * STRIPE, TAIL)
    pltpu.sync_copy(accum.at[tl], stage.at[pl.ds(0, TAIL)])
    pltpu.sync_copy(stage.at[pl.ds(0, TAIL)], out.at[tl])


def _agg_body(hA, eiA, hB, eiB,
              sumA, sumB, accum, idx0, idx1, idx2, idx3, rows0, rows1, stage,
              isem0, isem1, isem2, isem3, gsem0, gsem1, wsem0, wsem1):
  cid = lax.axis_index("c")
  sid = lax.axis_index("s")
  _zero_accum(accum, stage, sid)
  plsc.subcore_barrier()

  idxs = (idx0, idx1, idx2, idx3)
  isems = (isem0, isem1, isem2, isem3)
  rowss = (rows0, rows1)
  gsems = (gsem0, gsem1)
  wsems = (wsem0, wsem1)

  def run(ei, h, sum_out):
    def active(i):
      return sid + NSUB * i < NCT

    def start_idx(q, i):
      off = (sid + NSUB * i) * CH
      pltpu.make_async_copy(ei.at[:, pl.ds(off, CH)], idxs[q],
                            isems[q]).start()

    def wait_idx(q):
      pltpu.make_async_copy(ei.at[:, pl.ds(0, CH)], idxs[q], isems[q]).wait()

    def start_gather(b, q):
      pltpu.make_async_copy(h.at[idxs[q].at[0]], rowss[b], gsems[b]).start()

    def wait_gather(b, q):
      pltpu.make_async_copy(h.at[idxs[q].at[0]], rowss[b], gsems[b]).wait()

    def start_scatter(b, q):
      pltpu.make_async_copy(rowss[b], accum.at[idxs[q].at[1]],
                            wsems[b]).start(add=True)

    def wait_scatter(b, q):
      pltpu.make_async_copy(rowss[b], accum.at[idxs[q].at[1]],
                            wsems[b]).wait()

    start_idx(0, 0)
    start_idx(1, 1)
    start_idx(2, 2)
    wait_idx(0)
    start_gather(0, 0)

    def step(k, _):
      for b4 in range(4):
        j = 4 * k + b4
        b = b4 % 2
        o = 1 - b
        qj = b4
        qp = (b4 - 1) % 4
        qn = (b4 + 1) % 4
        qn3 = (b4 + 3) % 4
        @pl.when((j >= 1) & active(j - 1))
        def _():
          wait_scatter(o, qp)
        @pl.when(active(j + 1))
        def _():
          wait_idx(qn)
          start_gather(o, qn)
        @pl.when(active(j))
        def _():
          wait_gather(b, qj)
          start_scatter(b, qj)
        @pl.when(active(j + 3))
        def _():
          start_idx(qn3, j + 3)
      return 0
    lax.fori_loop(0, (ITERS + 4) // 4, step, 0)

    plsc.subcore_barrier()
    _write_out(accum, stage, sid, sum_out)

  @pl.when(cid == 0)
  def _():
    run(eiA, hA, sumA)

  @pl.when(cid == 1)
  def _():
    run(eiB, hB, sumB)


def _cnt_body(eiA, eiB, cntA, cntB,
              accum, idx0, idx1, idx2, idx3, ones_v, stage,
              isem0, isem1, isem2, isem3, wsem0, wsem1):
  cid = lax.axis_index("c")
  sid = lax.axis_index("s")
  _zero_accum(accum, stage, sid)
  one16 = jnp.ones((16,), jnp.float32)
  def ob(i, _):
    for j in range(D // 16):
      ones_v[i, pl.ds(j * 16, 16)] = one16
    return 0
  lax.fori_loop(0, CH, ob, 0)
  plsc.subcore_barrier()

  idxs = (idx0, idx1, idx2, idx3)
  isems = (isem0, isem1, isem2, isem3)
  wsems = (wsem0, wsem1)

  def run(ei, cnt_out):
    def active(i):
      return sid + NSUB * i < NCT

    def start_idx(q, i):
      off = (sid + NSUB * i) * CH
      pltpu.make_async_copy(ei.at[:, pl.ds(off, CH)], idxs[q],
                            isems[q]).start()

    def wait_idx(q):
      pltpu.make_async_copy(ei.at[:, pl.ds(0, CH)], idxs[q], isems[q]).wait()

    def start_scatter(b, q):
      pltpu.make_async_copy(ones_v, accum.at[idxs[q].at[1]],
                            wsems[b]).start(add=True)

    def wait_scatter(b, q):
      pltpu.make_async_copy(ones_v, accum.at[idxs[q].at[1]],
                            wsems[b]).wait()

    start_idx(0, 0)
    start_idx(1, 1)

    def step(k, _):
      for b4 in range(4):
        j = 4 * k + b4
        b = b4 % 2
        o = 1 - b
        qj = b4
        qp = (b4 - 1) % 4
        qn2 = (b4 + 2) % 4
        @pl.when(active(j))
        def _():
          wait_idx(qj)
          start_scatter(b, qj)
        @pl.when((j >= 1) & active(j - 1))
        def _():
          wait_scatter(o, qp)
        @pl.when(active(j + 2))
        def _():
          start_idx(qn2, j + 2)
      return 0
    lax.fori_loop(0, (ITERS + 4) // 4, step, 0)

    plsc.subcore_barrier()
    _write_out(accum, stage, sid, cnt_out)

  @pl.when(cid == 0)
  def _():
    run(eiA, cntA)

  @pl.when(cid == 1)
  def _():
    run(eiB, cntB)


_SC_MESH = plsc.VectorSubcoreMesh(core_axis_name="c", subcore_axis_name="s")

_agg = pl.kernel(
    _agg_body,
    out_type=(jax.ShapeDtypeStruct((N, D), jnp.float32),
              jax.ShapeDtypeStruct((N, D), jnp.float32)),
    mesh=_SC_MESH,
    scratch_types=(
        pltpu.VMEM_SHARED((N, D), jnp.float32),
        pltpu.VMEM((2, CH), jnp.int32),
        pltpu.VMEM((2, CH), jnp.int32),
        pltpu.VMEM((2, CH), jnp.int32),
        pltpu.VMEM((2, CH), jnp.int32),
        pltpu.VMEM((CH, D), jnp.float32),
        pltpu.VMEM((CH, D), jnp.float32),
        pltpu.VMEM((STG, D), jnp.float32),
        pltpu.SemaphoreType.DMA,
        pltpu.SemaphoreType.DMA,
        pltpu.SemaphoreType.DMA,
        pltpu.SemaphoreType.DMA,
        pltpu.SemaphoreType.DMA,
        pltpu.SemaphoreType.DMA,
        pltpu.SemaphoreType.DMA,
        pltpu.SemaphoreType.DMA,
    ))

_cnt = pl.kernel(
    _cnt_body,
    out_type=(jax.ShapeDtypeStruct((N, D), jnp.float32),
              jax.ShapeDtypeStruct((N, D), jnp.float32)),
    mesh=_SC_MESH,
    scratch_types=(
        pltpu.VMEM_SHARED((N, D), jnp.float32),
        pltpu.VMEM((2, CH), jnp.int32),
        pltpu.VMEM((2, CH), jnp.int32),
        pltpu.VMEM((2, CH), jnp.int32),
        pltpu.VMEM((2, CH), jnp.int32),
        pltpu.VMEM((CH, D), jnp.float32),
        pltpu.VMEM((STG, D), jnp.float32),
        pltpu.SemaphoreType.DMA,
        pltpu.SemaphoreType.DMA,
        pltpu.SemaphoreType.DMA,
        pltpu.SemaphoreType.DMA,
        pltpu.SemaphoreType.DMA,
        pltpu.SemaphoreType.DMA,
    ))


BR = 400


def _enc_body(x_ref, w_ref, b_ref, o_ref):
  o_ref[...] = (jnp.dot(x_ref[...], w_ref[...],
                        preferred_element_type=jnp.float32) + b_ref[...])


def _enc(x, W, b):
  return pl.pallas_call(
      _enc_body,
      grid=(N // BR,),
      in_specs=[pl.BlockSpec((BR, D), lambda i: (i, 0)),
                pl.BlockSpec((D, D), lambda i: (0, 0)),
                pl.BlockSpec((1, D), lambda i: (0, 0))],
      out_specs=pl.BlockSpec((BR, D), lambda i: (i, 0)),
      out_shape=jax.ShapeDtypeStruct((N, D), jnp.float32),
  )(x, W, b.reshape(1, D))


def _layer_body(s_ref, c_ref, h_ref, wl_ref, bl_ref, wr_ref, o_ref):
  mean = s_ref[...] / jnp.maximum(c_ref[...], 1.0)
  o_ref[...] = (jnp.dot(mean, wl_ref[...], preferred_element_type=jnp.float32)
                + jnp.dot(h_ref[...], wr_ref[...],
                          preferred_element_type=jnp.float32)
                + bl_ref[...])


def _layer(s, cnt_col, h, Wl, bl, Wr):
  return pl.pallas_call(
      _layer_body,
      grid=(N // BR,),
      in_specs=[pl.BlockSpec((BR, D), lambda i: (i, 0)),
                pl.BlockSpec((BR, 1), lambda i: (i, 0)),
                pl.BlockSpec((BR, D), lambda i: (i, 0)),
                pl.BlockSpec((D, D), lambda i: (0, 0)),
                pl.BlockSpec((1, D), lambda i: (0, 0)),
                pl.BlockSpec((D, D), lambda i: (0, 0))],
      out_specs=pl.BlockSpec((BR, D), lambda i: (i, 0)),
      out_shape=jax.ShapeDtypeStruct((N, D), jnp.float32),
  )(s, cnt_col, h, Wl, bl.reshape(1, D), Wr)


def kernel(x_user, x_item, edge_index_ui, edge_index_iu,
           W_enc_user, b_enc_user, W_enc_item, b_enc_item,
           Wl0_ui, bl0_ui, Wr0_ui, Wl0_iu, bl0_iu, Wr0_iu,
           Wl1_ui, bl1_ui, Wr1_ui, Wl1_iu, bl1_iu, Wr1_iu):
  hu = _enc(x_user, W_enc_user, b_enc_user)
  hi = _enc(x_item, W_enc_item, b_enc_item)

  c_ui, c_iu = _cnt(edge_index_ui, edge_index_iu)
  cu = c_ui[:, :1]
  ci = c_iu[:, :1]

  s_ui, s_iu = _agg(hu, edge_index_ui, hi, edge_index_iu)
  ni = _layer(s_ui, cu, hi, Wl0_ui, bl0_ui, Wr0_ui)
  nu = _layer(s_iu, ci, hu, Wl0_iu, bl0_iu, Wr0_iu)

  s_ui1, s_iu1 = _agg(nu, edge_index_ui, ni, edge_index_iu)
  ni2 = _layer(s_ui1, cu, ni, Wl1_ui, bl1_ui, Wr1_ui)
  nu2 = _layer(s_iu1, ci, nu, Wl1_iu, bl1_iu, Wr1_iu)
  return (nu2, ni2)


# speedup vs baseline: 11.1843x; 1.1965x over previous
import jax
import jax.numpy as jnp
from jax import lax
from jax.experimental import pallas as pl
from jax.experimental.pallas import tpu as pltpu
from jax.experimental.pallas import tpu_sc as plsc

N = 10000
D = 128
E = 320000
CH = 128
NSUB = 16
NCT = E // CH
ITERS = -(-NCT // NSUB)
STRIPE = 624
TAIL = N - NSUB * STRIPE
HR = 80


def _zero_accum(accum, stage, sid):
  stg = stage.shape[0]
  zero16 = jnp.zeros((16,), jnp.float32)
  def zs(i, _):
    for j in range(D // 16):
      stage[i, pl.ds(j * 16, 16)] = zero16
    return 0
  lax.fori_loop(0, stg, zs, 0)
  def za(i, _):
    pltpu.sync_copy(stage, accum.at[pl.ds(sid * STRIPE + i * stg, stg)])
    return 0
  lax.fori_loop(0, STRIPE // stg, za, 0)
  @pl.when(sid == NSUB - 1)
  def _():
    pltpu.sync_copy(stage.at[pl.ds(0, TAIL)],
                    accum.at[pl.ds(NSUB * STRIPE, TAIL)])


def _write_out(accum, stage, sid, out):
  stg = stage.shape[0]
  def wo(i, _):
    sl = pl.ds(sid * STRIPE + i * stg, stg)
    pltpu.sync_copy(accum.at[sl], stage)
    pltpu.sync_copy(stage, out.at[sl])
    return 0
  lax.fori_loop(0, STRIPE // stg, wo, 0)
  @pl.when(sid == NSUB - 1)
  def _():
    tl = pl.ds(NSUB * STRIPE, TAIL)
    pltpu.sync_copy(accum.at[tl], stage.at[pl.ds(0, TAIL)])
    pltpu.sync_copy(stage.at[pl.ds(0, TAIL)], out.at[tl])


def _make_agg_body(with_hist):
  def body(hA, eiA, hB, eiB, *rest):
    if with_hist:
      (sumA, sumB, cntA, cntB, accum, idx0, idx1, idx2, idx3, rows0, rows1,
       stage, hist, isem0, isem1, isem2, isem3, gsem0, gsem1,
       wsem0, wsem1) = rest
    else:
      (sumA, sumB, accum, idx0, idx1, idx2, idx3, rows0, rows1,
       stage, isem0, isem1, isem2, isem3, gsem0, gsem1, wsem0, wsem1) = rest
      cntA = cntB = hist = None

    cid = lax.axis_index("c")
    sid = lax.axis_index("s")
    zero16 = jnp.zeros((16,), jnp.float32)
    _zero_accum(accum, stage, sid)
    if with_hist:
      def zh(i, _):
        for c in range(D // 16):
          hist[i, pl.ds(c * 16, 16)] = zero16
        return 0
      lax.fori_loop(0, HR, zh, 0)
    plsc.subcore_barrier()

    idxs = (idx0, idx1, idx2, idx3)
    isems = (isem0, isem1, isem2, isem3)
    rowss = (rows0, rows1)
    gsems = (gsem0, gsem1)
    wsems = (wsem0, wsem1)

    def run(ei, h, sum_out, cnt_out):
      def active(i):
        return sid + NSUB * i < NCT

      def start_idx(q, i):
        off = (sid + NSUB * i) * CH
        pltpu.make_async_copy(ei.at[:, pl.ds(off, CH)], idxs[q],
                              isems[q]).start()

      def wait_idx(q):
        pltpu.make_async_copy(ei.at[:, pl.ds(0, CH)], idxs[q],
                              isems[q]).wait()

      def start_gather(b, q):
        pltpu.make_async_copy(h.at[idxs[q].at[0]], rowss[b],
                              gsems[b]).start()

      def wait_gather(b, q):
        pltpu.make_async_copy(h.at[idxs[q].at[0]], rowss[b], gsems[b]).wait()

      def start_scatter(b, q):
        pltpu.make_async_copy(rowss[b], accum.at[idxs[q].at[1]],
                              wsems[b]).start(add=True)

      def wait_scatter(b, q):
        pltpu.make_async_copy(rowss[b], accum.at[idxs[q].at[1]],
                              wsems[b]).wait()

      start_idx(0, 0)
      start_idx(1, 1)
      start_idx(2, 2)
      wait_idx(0)
      start_gather(0, 0)

      def step(k, _):
        for b4 in range(4):
          j = 4 * k + b4
          b = b4 % 2
          o = 1 - b
          qj = b4
          qp = (b4 - 1) % 4
          qn = (b4 + 1) % 4
          qn3 = (b4 + 3) % 4
          @pl.when((j >= 1) & active(j - 1))
          def _():
            wait_scatter(o, qp)
          @pl.when(active(j + 1))
          def _():
            wait_idx(qn)
            start_gather(o, qn)
          @pl.when(active(j))
          def _():
            wait_gather(b, qj)
            start_scatter(b, qj)
            if with_hist:
              dq = idxs[qj]
              for l in range(CH // 16):
                dv = dq[1, pl.ds(l * 16, 16)]
                c, last = plsc.scan_count(dv)
                rdx = dv >> 7
                cdx = dv & 127
                plsc.addupdate_scatter(hist, [rdx, cdx],
                                       c.astype(jnp.float32), mask=last)
          @pl.when(active(j + 3))
          def _():
            start_idx(qn3, j + 3)
        return 0
      lax.fori_loop(0, (ITERS + 4) // 4, step, 0)

      plsc.subcore_barrier()
      _write_out(accum, stage, sid, sum_out)

      if with_hist:
        plsc.subcore_barrier()
        pltpu.sync_copy(hist, accum.at[pl.ds(sid * HR, HR)])
        plsc.subcore_barrier()
        @pl.when(sid < HR // 8)
        def _():
          def zr(r, _):
            for c in range(D // 16):
              rows1[r, pl.ds(c * 16, 16)] = zero16
            return 0
          lax.fori_loop(0, 8, zr, 0)
          def red(t, _):
            pltpu.sync_copy(accum.at[pl.ds(t * HR + sid * 8, 8)],
                            rows0.at[pl.ds(0, 8)])
            def addr(r, _):
              for c in range(D // 16):
                sl = pl.ds(c * 16, 16)
                rows1[r, sl] = rows1[r, sl] + rows0[r, sl]
              return 0
            lax.fori_loop(0, 8, addr, 0)
            return 0
          lax.fori_loop(0, NSUB, red, 0)
          pltpu.sync_copy(rows1.at[pl.ds(0, 8)],
                          cnt_out.at[pl.ds(sid * 8, 8)])

    @pl.when(cid == 0)
    def _():
      run(eiA, hA, sumA, cntA)

    @pl.when(cid == 1)
    def _():
      run(eiB, hB, sumB, cntB)

  return body


_SC_MESH = plsc.VectorSubcoreMesh(core_axis_name="c", subcore_axis_name="s")

_COMMON_SCRATCH = (
    pltpu.VMEM((2, CH), jnp.int32),
    pltpu.VMEM((2, CH), jnp.int32),
    pltpu.VMEM((2, CH), jnp.int32),
    pltpu.VMEM((2, CH), jnp.int32),
    pltpu.VMEM((CH, D), jnp.float32),
    pltpu.VMEM((CH, D), jnp.float32),
)
_SEMS = (
    pltpu.SemaphoreType.DMA,
    pltpu.SemaphoreType.DMA,
    pltpu.SemaphoreType.DMA,
    pltpu.SemaphoreType.DMA,
    pltpu.SemaphoreType.DMA,
    pltpu.SemaphoreType.DMA,
    pltpu.SemaphoreType.DMA,
    pltpu.SemaphoreType.DMA,
)

_agg = pl.kernel(
    _make_agg_body(False),
    out_type=(jax.ShapeDtypeStruct((N, D), jnp.float32),
              jax.ShapeDtypeStruct((N, D), jnp.float32)),
    mesh=_SC_MESH,
    scratch_types=(
        (pltpu.VMEM_SHARED((N, D), jnp.float32),)
        + _COMMON_SCRATCH
        + (pltpu.VMEM((48, D), jnp.float32),)
        + _SEMS))

_agg_hist = pl.kernel(
    _make_agg_body(True),
    out_type=(jax.ShapeDtypeStruct((N, D), jnp.float32),
              jax.ShapeDtypeStruct((N, D), jnp.float32),
              jax.ShapeDtypeStruct((HR, D), jnp.float32),
              jax.ShapeDtypeStruct((HR, D), jnp.float32)),
    mesh=_SC_MESH,
    scratch_types=(
        (pltpu.VMEM_SHARED((N, D), jnp.float32),)
        + _COMMON_SCRATCH
        + (pltpu.VMEM((16, D), jnp.float32),
           pltpu.VMEM((HR, D), jnp.float32))
        + _SEMS),
    compiler_params=pltpu.CompilerParams(needs_layout_passes=False))


BR = 400


def _enc_body(x_ref, w_ref, b_ref, o_ref):
  o_ref[...] = (jnp.dot(x_ref[...], w_ref[...],
                        preferred_element_type=jnp.float32) + b_ref[...])


def _enc(x, W, b):
  return pl.pallas_call(
      _enc_body,
      grid=(N // BR,),
      in_specs=[pl.BlockSpec((BR, D), lambda i: (i, 0)),
                pl.BlockSpec((D, D), lambda i: (0, 0)),
                pl.BlockSpec((1, D), lambda i: (0, 0))],
      out_specs=pl.BlockSpec((BR, D), lambda i: (i, 0)),
      out_shape=jax.ShapeDtypeStruct((N, D), jnp.float32),
  )(x, W, b.reshape(1, D))


def _layer_body(s_ref, c_ref, h_ref, wl_ref, bl_ref, wr_ref, o_ref):
  mean = s_ref[...] / jnp.maximum(c_ref[...], 1.0)
  o_ref[...] = (jnp.dot(mean, wl_ref[...], preferred_element_type=jnp.float32)
                + jnp.dot(h_ref[...], wr_ref[...],
                          preferred_element_type=jnp.float32)
                + bl_ref[...])


def _layer(s, cnt_col, h, Wl, bl, Wr):
  return pl.pallas_call(
      _layer_body,
      grid=(N // BR,),
      in_specs=[pl.BlockSpec((BR, D), lambda i: (i, 0)),
                pl.BlockSpec((BR, 1), lambda i: (i, 0)),
                pl.BlockSpec((BR, D), lambda i: (i, 0)),
                pl.BlockSpec((D, D), lambda i: (0, 0)),
                pl.BlockSpec((1, D), lambda i: (0, 0)),
                pl.BlockSpec((D, D), lambda i: (0, 0))],
      out_specs=pl.BlockSpec((BR, D), lambda i: (i, 0)),
      out_shape=jax.ShapeDtypeStruct((N, D), jnp.float32),
  )(s, cnt_col, h, Wl, bl.reshape(1, D), Wr)


def kernel(x_user, x_item, edge_index_ui, edge_index_iu,
           W_enc_user, b_enc_user, W_enc_item, b_enc_item,
           Wl0_ui, bl0_ui, Wr0_ui, Wl0_iu, bl0_iu, Wr0_iu,
           Wl1_ui, bl1_ui, Wr1_ui, Wl1_iu, bl1_iu, Wr1_iu):
  hu = _enc(x_user, W_enc_user, b_enc_user)
  hi = _enc(x_item, W_enc_item, b_enc_item)

  s_ui, s_iu, c_ui, c_iu = _agg_hist(hu, edge_index_ui, hi, edge_index_iu)
  cu = c_ui.reshape(HR * D, 1)[:N]
  ci = c_iu.reshape(HR * D, 1)[:N]
  ni = _layer(s_ui, cu, hi, Wl0_ui, bl0_ui, Wr0_ui)
  nu = _layer(s_iu, ci, hu, Wl0_iu, bl0_iu, Wr0_iu)

  s_ui1, s_iu1 = _agg(nu, edge_index_ui, ni, edge_index_iu)
  ni2 = _layer(s_ui1, cu, ni, Wl1_ui, bl1_ui, Wr1_ui)
  nu2 = _layer(s_iu1, ci, nu, Wl1_iu, bl1_iu, Wr1_iu)
  return (nu2, ni2)
